# baseline (speedup 1.0000x reference)
import functools

import jax
import jax.numpy as jnp
from jax import lax
from jax.experimental import pallas as pl
from jax.experimental.pallas import tpu as pltpu
from jax.experimental.pallas import tpu_sc as plsc

BATCH = 16384
EMBED_DIM = 32
LANES = 16
NUM_CORES = 2
NUM_SUBCORES = 16
NUM_WORKERS = NUM_CORES * NUM_SUBCORES
BPW = BATCH // NUM_WORKERS
LANE_TILE = 128
CHUNKS = BPW // LANES


def _build():
    mesh = plsc.VectorSubcoreMesh(core_axis_name="c", subcore_axis_name="s")

    @functools.partial(
        pl.kernel,
        mesh=mesh,
        out_type=jax.ShapeDtypeStruct((BATCH,), jnp.float32),
        compiler_params=pltpu.CompilerParams(
            needs_layout_passes=False, use_tc_tiling_on_sc=True),
        scratch_types=[
            pltpu.VMEM((BPW,), jnp.int32),
            pltpu.VMEM((BPW,), jnp.int32),
            pltpu.VMEM((24 * EMBED_DIM, LANE_TILE), jnp.float32),
            pltpu.VMEM((EMBED_DIM * LANES,), jnp.float32),
            pltpu.VMEM((EMBED_DIM * LANES,), jnp.float32),
            pltpu.VMEM((BPW,), jnp.float32),
            pltpu.SemaphoreType.DMA,
            pltpu.SemaphoreType.DMA,
        ],
    )
    def cmf_kernel(uidx_hbm, iidx_hbm, utab_hbm, ttab_hbm, out_hbm,
                   uidx_v, iidx_v, tiles_v, ucomp_v, tcomp_v, out_v,
                   sem_u, sem_i):
        wid = lax.axis_index("s") * NUM_CORES + lax.axis_index("c")
        base = wid * BPW

        pltpu.sync_copy(uidx_hbm.at[pl.ds(base, BPW)], uidx_v)
        pltpu.sync_copy(iidx_hbm.at[pl.ds(base, BPW)], iidx_v)

        iota16 = lax.iota(jnp.int32, LANES)
        slot_rows_u = iota16 * EMBED_DIM
        slot_rows_i = jnp.where(iota16 < 8,
                                (16 + iota16) * EMBED_DIM,
                                (iota16 - 8) * EMBED_DIM)
        lane_mod = jnp.full((LANES,), LANE_TILE, jnp.int32)

        def fire(tab_hbm, vec, k, slot, sem):
            tcol = vec[k] // LANE_TILE
            start = pl.multiple_of(tcol * LANE_TILE, LANE_TILE)
            return pltpu.async_copy(
                tab_hbm.at[:, pl.ds(start, LANE_TILE)],
                tiles_v.at[pl.ds(slot * EMBED_DIM, EMBED_DIM)], sem)

        def extract(comp_ref, vec, slot_rows):
            lane = lax.rem(vec, lane_mod)
            for d in range(EMBED_DIM):
                vals = plsc.load_gather(tiles_v, [slot_rows + d, lane])
                comp_ref[pl.ds(d * LANES, LANES)] = vals

        def chunk_body(c, _):
            uvec = uidx_v[pl.ds(c * LANES, LANES)]
            ivec = iidx_v[pl.ds(c * LANES, LANES)]
            ucopies = [fire(utab_hbm, uvec, k, k, sem_u)
                       for k in range(LANES)]
            icopies = [fire(ttab_hbm, ivec, k, 16 + k, sem_i)
                       for k in range(8)]
            for cp in ucopies:
                cp.wait()
            extract(ucomp_v, uvec, slot_rows_u)
            icopies += [fire(ttab_hbm, ivec, k, k - 8, sem_i)
                        for k in range(8, LANES)]
            for cp in icopies:
                cp.wait()
            extract(tcomp_v, ivec, slot_rows_i)
            acc = jnp.zeros((LANES,), jnp.float32)
            for d in range(EMBED_DIM):
                u = ucomp_v[pl.ds(d * LANES, LANES)]
                v = tcomp_v[pl.ds(d * LANES, LANES)]
                acc = acc + u * v
            out_v[pl.ds(c * LANES, LANES)] = 1.0 / (1.0 + jnp.exp(-acc))
            return 0

        lax.fori_loop(0, CHUNKS, chunk_body, 0)

        pltpu.sync_copy(out_v, out_hbm.at[pl.ds(base, BPW)])

    return cmf_kernel


@functools.cache
def _get_cmf():
    return _build()


def kernel(user_indices, item_indices, user_table, src_item_table, tgt_item_table):
    del src_item_table
    return _get_cmf()(user_indices.astype(jnp.int32),
                      item_indices.astype(jnp.int32),
                      user_table.T, tgt_item_table.T)


# speedup vs baseline: 1.1610x; 1.1610x over previous
import functools

import jax
import jax.numpy as jnp
from jax import lax
from jax.experimental import pallas as pl
from jax.experimental.pallas import tpu as pltpu
from jax.experimental.pallas import tpu_sc as plsc

BATCH = 16384
EMBED_DIM = 32
LANES = 16
NUM_CORES = 2
NUM_SUBCORES = 16
NUM_WORKERS = NUM_CORES * NUM_SUBCORES
BPW = BATCH // NUM_WORKERS
LANE_TILE = 128
CHUNKS = BPW // LANES


def _build():
    mesh = plsc.VectorSubcoreMesh(core_axis_name="c", subcore_axis_name="s")

    @functools.partial(
        pl.kernel,
        mesh=mesh,
        out_type=jax.ShapeDtypeStruct((BATCH,), jnp.float32),
        compiler_params=pltpu.CompilerParams(
            needs_layout_passes=False, use_tc_tiling_on_sc=True),
        scratch_types=[
            pltpu.VMEM((BPW,), jnp.int32),
            pltpu.VMEM((BPW,), jnp.int32),
            pltpu.VMEM((24 * EMBED_DIM, LANE_TILE), jnp.float32),
            pltpu.VMEM((EMBED_DIM * LANES,), jnp.float32),
            pltpu.VMEM((EMBED_DIM * LANES,), jnp.float32),
            pltpu.VMEM((BPW,), jnp.float32),
            pltpu.SemaphoreType.DMA,
            pltpu.SemaphoreType.DMA,
        ],
    )
    def cmf_kernel(uidx_hbm, iidx_hbm, utab_hbm, ttab_hbm, out_hbm,
                   uidx_v, iidx_v, tiles_v, ucomp_v, tcomp_v, out_v,
                   sem_u, sem_i):
        wid = lax.axis_index("s") * NUM_CORES + lax.axis_index("c")
        base = wid * BPW

        pltpu.sync_copy(uidx_hbm.at[pl.ds(base, BPW)], uidx_v)
        pltpu.sync_copy(iidx_hbm.at[pl.ds(base, BPW)], iidx_v)

        iota16 = lax.iota(jnp.int32, LANES)
        slot_rows = iota16 * EMBED_DIM
        lane_mod = jnp.full((LANES,), LANE_TILE, jnp.int32)

        def fetch_and_extract(tab_hbm, idx_ref, comp_ref, c, sem):
            vec = idx_ref[pl.ds(c * LANES, LANES)]
            copies = []
            for k in range(LANES):
                tcol = vec[k] // LANE_TILE
                start = pl.multiple_of(tcol * LANE_TILE, LANE_TILE)
                copies.append(pltpu.async_copy(
                    tab_hbm.at[:, pl.ds(start, LANE_TILE)],
                    tiles_v.at[pl.ds(k * EMBED_DIM, EMBED_DIM)], sem))
            for cp in copies:
                cp.wait()
            lane = lax.rem(vec, lane_mod)
            for d in range(EMBED_DIM):
                vals = plsc.load_gather(tiles_v, [slot_rows + d, lane])
                comp_ref[pl.ds(d * LANES, LANES)] = vals

        def chunk_body(c, _):
            fetch_and_extract(utab_hbm, uidx_v, ucomp_v, c, sem_u)
            fetch_and_extract(ttab_hbm, iidx_v, tcomp_v, c, sem_i)
            acc = jnp.zeros((LANES,), jnp.float32)
            for d in range(EMBED_DIM):
                u = ucomp_v[pl.ds(d * LANES, LANES)]
                v = tcomp_v[pl.ds(d * LANES, LANES)]
                acc = acc + u * v
            out_v[pl.ds(c * LANES, LANES)] = 1.0 / (1.0 + jnp.exp(-acc))
            return 0

        lax.fori_loop(0, CHUNKS, chunk_body, 0)

        pltpu.sync_copy(out_v, out_hbm.at[pl.ds(base, BPW)])

    return cmf_kernel


@functools.cache
def _get_cmf():
    return _build()


def kernel(user_indices, item_indices, user_table, src_item_table, tgt_item_table):
    del src_item_table
    return _get_cmf()(user_indices.astype(jnp.int32),
                      item_indices.astype(jnp.int32),
                      user_table.T, tgt_item_table.T)
